# baseline (speedup 1.0000x reference)
import jax
import jax.numpy as jnp
from jax import lax
from jax.experimental import pallas as pl
from jax.experimental.pallas import tpu as pltpu
from jax.experimental.pallas import tpu_sc as plsc

N = 10000
E = 320000
D = 128
NC = 2
NS = 16
L = 16
NW = NC * NS
ESUB = 128

NPAD = 10240
RSTRIDE = NPAD // NS

EC = 256
NCHUNK = E // EC
KMAX = (NCHUNK + NW - 1) // NW

DC = 512
DNCHUNK = E // DC
DKMAX = (DNCHUNK + NW - 1) // NW

CC = 128
NCC = NPAD // CC
NFULL = N // CC
NREM = N - NFULL * CC
CKMAX = (NCC + NW - 1) // NW

_SC_MESH = dict(core_axis_name="c", subcore_axis_name="s",
                num_cores=NC, num_subcores=NS)


def _mm3_body(x_ref, w1_ref, b1_ref, w2_ref, b2_ref, ww_ref, bw_ref,
              g1_ref, g2_ref, gw_ref):
    x = x_ref[...]
    dn = (((1,), (1,)), ((), ()))
    g1_ref[...] = lax.dot_general(
        x, w1_ref[...], dn, preferred_element_type=jnp.float32) + b1_ref[...]
    g2_ref[...] = lax.dot_general(
        x, w2_ref[...], dn, preferred_element_type=jnp.float32) + b2_ref[...]
    gw_ref[...] = lax.dot_general(
        x, ww_ref[...], dn, preferred_element_type=jnp.float32) + bw_ref[...]


def _mm3(ace, W1, b1, W2, b2, Ww, bw):
    RB = 1000
    wspec = pl.BlockSpec((D, D), lambda i: (0, 0))
    bspec = pl.BlockSpec((1, D), lambda i: (0, 0))
    xspec = pl.BlockSpec((RB, D), lambda i: (i, 0))
    return pl.pallas_call(
        _mm3_body,
        grid=(N // RB,),
        in_specs=[xspec, wspec, bspec, wspec, bspec, wspec, bspec],
        out_specs=[xspec, xspec, xspec],
        out_shape=[jax.ShapeDtypeStruct((N, D), jnp.float32)] * 3,
    )(ace, W1, b1.reshape(1, D), W2, b2.reshape(1, D), Ww, bw.reshape(1, D))


def _edge_body(acew_hbm, nbr_hbm, idx1_hbm, ew_hbm, aggr_hbm,
               acc, nbr_v, idx1_v, ew_v, rows_v, sem):
    c = lax.axis_index("c")
    s = lax.axis_index("s")
    w = s * NC + c
    zero = jnp.zeros((L,), jnp.float32)

    def zrow(i, _):
        for j in range(D // L):
            rows_v[i, pl.ds(j * L, L)] = zero
        return 0

    lax.fori_loop(0, EC, zrow, 0)
    r0 = s * RSTRIDE
    for off in range(0, RSTRIDE, EC):
        sz = min(EC, RSTRIDE - off)
        pltpu.sync_copy(rows_v.at[pl.ds(0, sz)],
                        acc.at[pl.ds(r0 + off, sz)])
    plsc.subcore_barrier()

    def chunk(k, _):
        g = k * NW + w

        @pl.when(g < NCHUNK)
        def _():
            pltpu.sync_copy(nbr_hbm.at[g], nbr_v)
            pltpu.sync_copy(idx1_hbm.at[g], idx1_v)
            pltpu.sync_copy(ew_hbm.at[pl.ds(g * EC, EC)],
                            ew_v.at[pl.ds(0, EC)])
            descs = [
                pltpu.async_copy(acew_hbm.at[nbr_v.at[j]],
                                 rows_v.at[pl.ds(j * ESUB, ESUB)], sem)
                for j in range(EC // ESUB)
            ]
            for dsc in descs:
                dsc.wait()

            def scale(i, _):
                wv = ew_v[pl.ds(i, L)]
                dv = jnp.full((L,), wv[0], jnp.float32)
                for j in range(D // L):
                    sl = pl.ds(j * L, L)
                    rows_v[i, sl] = rows_v[i, sl] * dv
                return 0

            lax.fori_loop(0, EC, scale, 0)
            for j in range(EC // ESUB):
                pltpu.sync_copy(rows_v.at[pl.ds(j * ESUB, ESUB)],
                                acc.at[idx1_v.at[j]], add=True)
        return 0

    lax.fori_loop(0, KMAX, chunk, 0)
    plsc.subcore_barrier()
    pltpu.sync_copy(acc.at[pl.ds(r0, RSTRIDE)],
                    aggr_hbm.at[c].at[pl.ds(r0, RSTRIDE)])


def _edge_kernel(acew, nbr3, idx13, ew):
    mesh = plsc.VectorSubcoreMesh(**_SC_MESH)
    run = pl.kernel(
        _edge_body, mesh=mesh,
        out_type=jax.ShapeDtypeStruct((NC, NPAD, D), jnp.float32),
        scratch_types=[
            pltpu.MemorySpace.VMEM_SHARED((NPAD, D), jnp.float32),
            pltpu.MemorySpace.VMEM((EC // ESUB, ESUB), jnp.int32),
            pltpu.MemorySpace.VMEM((EC // ESUB, ESUB), jnp.int32),
            pltpu.MemorySpace.VMEM((EC + L,), jnp.float32),
            pltpu.MemorySpace.VMEM((EC, D), jnp.float32),
            pltpu.SemaphoreType.DMA,
        ],
    )
    return run(acew, nbr3, idx13, ew)


def _deg_body(idx1_hbm, ew_hbm, deg_hbm, dacc, idx1_v, ew_v):
    c = lax.axis_index("c")
    s = lax.axis_index("s")
    w = s * NC + c

    def zrow(i, _):
        ew_v[pl.ds(i * L, L)] = jnp.zeros((L,), jnp.float32)
        return 0

    lax.fori_loop(0, DC // L, zrow, 0)
    r0 = s * RSTRIDE
    for off in range(0, RSTRIDE, DC):
        sz = min(DC, RSTRIDE - off)
        pltpu.sync_copy(ew_v.at[pl.ds(0, sz)],
                        dacc.at[pl.ds(r0 + off, sz)])
    plsc.subcore_barrier()

    def chunk(k, _):
        g = k * NW + w

        @pl.when(g < DNCHUNK)
        def _():
            pltpu.sync_copy(idx1_hbm.at[g], idx1_v)
            pltpu.sync_copy(ew_hbm.at[pl.ds(g * DC, DC)], ew_v)
            for j in range(DC // ESUB):
                pltpu.sync_copy(ew_v.at[pl.ds(j * ESUB, ESUB)],
                                dacc.at[idx1_v.at[j]], add=True)
        return 0

    lax.fori_loop(0, DKMAX, chunk, 0)
    plsc.subcore_barrier()
    pltpu.sync_copy(dacc.at[pl.ds(r0, RSTRIDE)],
                    deg_hbm.at[pl.ds(c * NPAD + r0, RSTRIDE)])


def _deg_kernel(idx13, ew):
    mesh = plsc.VectorSubcoreMesh(**_SC_MESH)
    run = pl.kernel(
        _deg_body, mesh=mesh,
        out_type=jax.ShapeDtypeStruct((NC * NPAD,), jnp.float32),
        scratch_types=[
            pltpu.MemorySpace.VMEM_SHARED((NPAD,), jnp.float32),
            pltpu.MemorySpace.VMEM((DC // ESUB, ESUB), jnp.int32),
            pltpu.MemorySpace.VMEM((DC,), jnp.float32),
        ],
    )
    return run(idx13, ew)


def _combine_body(g1_hbm, g2_hbm, vn_hbm, aggr_hbm, deg_hbm, out_hbm,
                  vn_v, h1_v, h2_v, a0_v, a1_v, d0_v, d1_v, out_v, sem):
    c = lax.axis_index("c")
    s = lax.axis_index("s")
    w = s * NC + c

    def chunk(k, _):
        cid = k * NW + w

        @pl.when(cid < NCC)
        def _():
            r0 = cid * CC
            pltpu.sync_copy(vn_hbm.at[pl.ds(r0, CC)], vn_v)
            dsc1 = pltpu.async_copy(g1_hbm.at[vn_v], h1_v, sem)
            dsc2 = pltpu.async_copy(g2_hbm.at[vn_v], h2_v, sem)
            dsc1.wait()
            dsc2.wait()
            pltpu.sync_copy(aggr_hbm.at[0].at[pl.ds(r0, CC)], a0_v)
            pltpu.sync_copy(aggr_hbm.at[1].at[pl.ds(r0, CC)], a1_v)
            pltpu.sync_copy(deg_hbm.at[pl.ds(r0, CC)],
                            d0_v.at[pl.ds(0, CC)])
            pltpu.sync_copy(deg_hbm.at[pl.ds(NPAD + r0, CC)],
                            d1_v.at[pl.ds(0, CC)])

            def row(i, _):
                d0 = d0_v[pl.ds(i, L)]
                d1 = d1_v[pl.ds(i, L)]
                dv = jnp.full((L,), d0[0] + d1[0], jnp.float32)
                for j in range(D // L):
                    sl = pl.ds(j * L, L)
                    out_v[i, sl] = (dv * h1_v[i, sl]
                                    + (a0_v[i, sl] + a1_v[i, sl]
                                       + h2_v[i, sl]))
                return 0

            lax.fori_loop(0, CC, row, 0)

            @pl.when(cid < NFULL)
            def _():
                pltpu.sync_copy(out_v, out_hbm.at[pl.ds(r0, CC)])

            @pl.when(cid == NFULL)
            def _():
                pltpu.sync_copy(out_v.at[pl.ds(0, NREM)],
                                out_hbm.at[pl.ds(r0, NREM)])
        return 0

    lax.fori_loop(0, CKMAX, chunk, 0)


def _combine_kernel(g1, g2, vn, aggr, deg):
    mesh = plsc.VectorSubcoreMesh(**_SC_MESH)
    run = pl.kernel(
        _combine_body, mesh=mesh,
        out_type=jax.ShapeDtypeStruct((N, D), jnp.float32),
        scratch_types=[
            pltpu.MemorySpace.VMEM((CC,), jnp.int32),
            pltpu.MemorySpace.VMEM((CC, D), jnp.float32),
            pltpu.MemorySpace.VMEM((CC, D), jnp.float32),
            pltpu.MemorySpace.VMEM((CC, D), jnp.float32),
            pltpu.MemorySpace.VMEM((CC, D), jnp.float32),
            pltpu.MemorySpace.VMEM((CC + L,), jnp.float32),
            pltpu.MemorySpace.VMEM((CC + L,), jnp.float32),
            pltpu.MemorySpace.VMEM((CC, D), jnp.float32),
            pltpu.SemaphoreType.DMA,
        ],
    )
    return run(g1, g2, vn, aggr, deg)


def kernel(all_community_embeddings, memory, valid_nodes, index, index1,
           neighbors_unique, index_noself, index1_noself,
           neighbors_unique_noself, edge_weight, W1, b1, W2, b2, Ww, bw):
    del memory, index, index_noself, index1_noself, neighbors_unique_noself
    g1, g2, acew = _mm3(all_community_embeddings, W1, b1, W2, b2, Ww, bw)
    nbr3 = neighbors_unique.astype(jnp.int32).reshape(NCHUNK, EC // ESUB, ESUB)
    idx13 = index1.astype(jnp.int32).reshape(NCHUNK, EC // ESUB, ESUB)
    idx13d = index1.astype(jnp.int32).reshape(DNCHUNK, DC // ESUB, ESUB)
    aggr = _edge_kernel(acew, nbr3, idx13, edge_weight)
    deg = _deg_kernel(idx13d, edge_weight)
    vnp = jnp.pad(valid_nodes.astype(jnp.int32), (0, NPAD - N))
    return _combine_kernel(g1, g2, vnp, aggr, deg)


# speedup vs baseline: 1.1697x; 1.1697x over previous
import jax
import jax.numpy as jnp
from jax import lax
from jax.experimental import pallas as pl
from jax.experimental.pallas import tpu as pltpu
from jax.experimental.pallas import tpu_sc as plsc

N = 10000
E = 320000
D = 128
NC = 2
NS = 16
L = 16
NW = NC * NS
ESUB = 128

NPAD = 10240
RSTRIDE = NPAD // NS

EC = 128
NCHUNK = E // EC
KMAX = (NCHUNK + NW - 1) // NW
KMAX2 = (KMAX + 1) // 2

DC = 512
DNCHUNK = E // DC
DKMAX = (DNCHUNK + NW - 1) // NW

CC = 128
NCC = NPAD // CC
NFULL = N // CC
NREM = N - NFULL * CC
CKMAX = (NCC + NW - 1) // NW

_SC_MESH = dict(core_axis_name="c", subcore_axis_name="s",
                num_cores=NC, num_subcores=NS)


def _mm3_body(x_ref, w1_ref, b1_ref, w2_ref, b2_ref, ww_ref, bw_ref,
              g1_ref, g2_ref, gw_ref):
    x = x_ref[...]
    dn = (((1,), (1,)), ((), ()))
    g1_ref[...] = lax.dot_general(
        x, w1_ref[...], dn, preferred_element_type=jnp.float32) + b1_ref[...]
    g2_ref[...] = lax.dot_general(
        x, w2_ref[...], dn, preferred_element_type=jnp.float32) + b2_ref[...]
    gw_ref[...] = lax.dot_general(
        x, ww_ref[...], dn, preferred_element_type=jnp.float32) + bw_ref[...]


def _mm3(ace, W1, b1, W2, b2, Ww, bw):
    RB = 1000
    wspec = pl.BlockSpec((D, D), lambda i: (0, 0))
    bspec = pl.BlockSpec((1, D), lambda i: (0, 0))
    xspec = pl.BlockSpec((RB, D), lambda i: (i, 0))
    return pl.pallas_call(
        _mm3_body,
        grid=(N // RB,),
        in_specs=[xspec, wspec, bspec, wspec, bspec, wspec, bspec],
        out_specs=[xspec, xspec, xspec],
        out_shape=[jax.ShapeDtypeStruct((N, D), jnp.float32)] * 3,
    )(ace, W1, b1.reshape(1, D), W2, b2.reshape(1, D), Ww, bw.reshape(1, D))


def _edge_body(acew_hbm, pidx_hbm, ew_hbm, aggr_hbm,
               acc, idxs0_v, idxs1_v, ew0_v, ew1_v, rows0_v, rows1_v,
               sem0, sem1):
    c = lax.axis_index("c")
    s = lax.axis_index("s")
    w = s * NC + c
    zero = jnp.zeros((L,), jnp.float32)

    def zrow(i, _):
        for j in range(D // L):
            rows0_v[i, pl.ds(j * L, L)] = zero
        return 0

    lax.fori_loop(0, EC, zrow, 0)
    r0 = s * RSTRIDE
    for off in range(0, RSTRIDE, EC):
        pltpu.sync_copy(rows0_v, acc.at[pl.ds(r0 + off, EC)])
    plsc.subcore_barrier()

    def fetch(g, idxs_v, ew_v, rows_v, sem):
        pltpu.sync_copy(pidx_hbm.at[g], idxs_v)
        pltpu.sync_copy(ew_hbm.at[pl.ds(g * EC, EC)],
                        ew_v.at[pl.ds(0, EC)])
        pltpu.make_async_copy(acew_hbm.at[idxs_v.at[0]], rows_v, sem).start()

    def consume(idxs_v, ew_v, rows_v, sem):
        pltpu.make_async_copy(acew_hbm.at[idxs_v.at[0]], rows_v, sem).wait()

        def scale(i, _):
            wv = ew_v[pl.ds(i, L)]
            dv = jnp.full((L,), wv[0], jnp.float32)
            for j in range(D // L):
                sl = pl.ds(j * L, L)
                rows_v[i, sl] = rows_v[i, sl] * dv
            return 0

        lax.fori_loop(0, EC, scale, 0)
        pltpu.sync_copy(rows_v, acc.at[idxs_v.at[1]], add=True)

    fetch(w, idxs0_v, ew0_v, rows0_v, sem0)

    def chunk2(k2, _):
        ga = (2 * k2) * NW + w
        gb = ga + NW
        gc = gb + NW

        @pl.when(gb < NCHUNK)
        def _():
            fetch(gb, idxs1_v, ew1_v, rows1_v, sem1)

        @pl.when(ga < NCHUNK)
        def _():
            consume(idxs0_v, ew0_v, rows0_v, sem0)

        @pl.when(gc < NCHUNK)
        def _():
            fetch(gc, idxs0_v, ew0_v, rows0_v, sem0)

        @pl.when(gb < NCHUNK)
        def _():
            consume(idxs1_v, ew1_v, rows1_v, sem1)

        return 0

    lax.fori_loop(0, KMAX2, chunk2, 0)
    plsc.subcore_barrier()
    pltpu.sync_copy(acc.at[pl.ds(r0, RSTRIDE)],
                    aggr_hbm.at[c].at[pl.ds(r0, RSTRIDE)])


def _edge_kernel(acew, pidx, ew):
    mesh = plsc.VectorSubcoreMesh(**_SC_MESH)
    run = pl.kernel(
        _edge_body, mesh=mesh,
        out_type=jax.ShapeDtypeStruct((NC, NPAD, D), jnp.float32),
        scratch_types=[
            pltpu.MemorySpace.VMEM_SHARED((NPAD, D), jnp.float32),
            pltpu.MemorySpace.VMEM((2, ESUB), jnp.int32),
            pltpu.MemorySpace.VMEM((2, ESUB), jnp.int32),
            pltpu.MemorySpace.VMEM((EC + L,), jnp.float32),
            pltpu.MemorySpace.VMEM((EC + L,), jnp.float32),
            pltpu.MemorySpace.VMEM((EC, D), jnp.float32),
            pltpu.MemorySpace.VMEM((EC, D), jnp.float32),
            pltpu.SemaphoreType.DMA,
            pltpu.SemaphoreType.DMA,
        ],
    )
    return run(acew, pidx, ew)


def _deg_body(idx1_hbm, ew_hbm, deg_hbm, dacc, idx1_v, ew_v):
    c = lax.axis_index("c")
    s = lax.axis_index("s")
    w = s * NC + c

    def zrow(i, _):
        ew_v[pl.ds(i * L, L)] = jnp.zeros((L,), jnp.float32)
        return 0

    lax.fori_loop(0, DC // L, zrow, 0)
    r0 = s * RSTRIDE
    for off in range(0, RSTRIDE, DC):
        sz = min(DC, RSTRIDE - off)
        pltpu.sync_copy(ew_v.at[pl.ds(0, sz)],
                        dacc.at[pl.ds(r0 + off, sz)])
    plsc.subcore_barrier()

    def chunk(k, _):
        g = k * NW + w

        @pl.when(g < DNCHUNK)
        def _():
            pltpu.sync_copy(idx1_hbm.at[g], idx1_v)
            pltpu.sync_copy(ew_hbm.at[pl.ds(g * DC, DC)], ew_v)
            for j in range(DC // ESUB):
                pltpu.sync_copy(ew_v.at[pl.ds(j * ESUB, ESUB)],
                                dacc.at[idx1_v.at[j]], add=True)
        return 0

    lax.fori_loop(0, DKMAX, chunk, 0)
    plsc.subcore_barrier()
    pltpu.sync_copy(dacc.at[pl.ds(r0, RSTRIDE)],
                    deg_hbm.at[pl.ds(c * NPAD + r0, RSTRIDE)])


def _deg_kernel(idx13, ew):
    mesh = plsc.VectorSubcoreMesh(**_SC_MESH)
    run = pl.kernel(
        _deg_body, mesh=mesh,
        out_type=jax.ShapeDtypeStruct((NC * NPAD,), jnp.float32),
        scratch_types=[
            pltpu.MemorySpace.VMEM_SHARED((NPAD,), jnp.float32),
            pltpu.MemorySpace.VMEM((DC // ESUB, ESUB), jnp.int32),
            pltpu.MemorySpace.VMEM((DC,), jnp.float32),
        ],
    )
    return run(idx13, ew)


def _combine_body(g1_hbm, g2_hbm, vn_hbm, aggr_hbm, deg_hbm, out_hbm,
                  vn_v, h1_v, h2_v, a0_v, a1_v, d0_v, d1_v, out_v, sem):
    c = lax.axis_index("c")
    s = lax.axis_index("s")
    w = s * NC + c

    def chunk(k, _):
        cid = k * NW + w

        @pl.when(cid < NCC)
        def _():
            r0 = cid * CC
            pltpu.sync_copy(vn_hbm.at[pl.ds(r0, CC)], vn_v)
            dsc1 = pltpu.async_copy(g1_hbm.at[vn_v], h1_v, sem)
            dsc2 = pltpu.async_copy(g2_hbm.at[vn_v], h2_v, sem)
            dsc1.wait()
            dsc2.wait()
            pltpu.sync_copy(aggr_hbm.at[0].at[pl.ds(r0, CC)], a0_v)
            pltpu.sync_copy(aggr_hbm.at[1].at[pl.ds(r0, CC)], a1_v)
            pltpu.sync_copy(deg_hbm.at[pl.ds(r0, CC)],
                            d0_v.at[pl.ds(0, CC)])
            pltpu.sync_copy(deg_hbm.at[pl.ds(NPAD + r0, CC)],
                            d1_v.at[pl.ds(0, CC)])

            def row(i, _):
                d0 = d0_v[pl.ds(i, L)]
                d1 = d1_v[pl.ds(i, L)]
                dv = jnp.full((L,), d0[0] + d1[0], jnp.float32)
                for j in range(D // L):
                    sl = pl.ds(j * L, L)
                    out_v[i, sl] = (dv * h1_v[i, sl]
                                    + (a0_v[i, sl] + a1_v[i, sl]
                                       + h2_v[i, sl]))
                return 0

            lax.fori_loop(0, CC, row, 0)

            @pl.when(cid < NFULL)
            def _():
                pltpu.sync_copy(out_v, out_hbm.at[pl.ds(r0, CC)])

            @pl.when(cid == NFULL)
            def _():
                pltpu.sync_copy(out_v.at[pl.ds(0, NREM)],
                                out_hbm.at[pl.ds(r0, NREM)])
        return 0

    lax.fori_loop(0, CKMAX, chunk, 0)


def _combine_kernel(g1, g2, vn, aggr, deg):
    mesh = plsc.VectorSubcoreMesh(**_SC_MESH)
    run = pl.kernel(
        _combine_body, mesh=mesh,
        out_type=jax.ShapeDtypeStruct((N, D), jnp.float32),
        scratch_types=[
            pltpu.MemorySpace.VMEM((CC,), jnp.int32),
            pltpu.MemorySpace.VMEM((CC, D), jnp.float32),
            pltpu.MemorySpace.VMEM((CC, D), jnp.float32),
            pltpu.MemorySpace.VMEM((CC, D), jnp.float32),
            pltpu.MemorySpace.VMEM((CC, D), jnp.float32),
            pltpu.MemorySpace.VMEM((CC + L,), jnp.float32),
            pltpu.MemorySpace.VMEM((CC + L,), jnp.float32),
            pltpu.MemorySpace.VMEM((CC, D), jnp.float32),
            pltpu.SemaphoreType.DMA,
        ],
    )
    return run(g1, g2, vn, aggr, deg)


def kernel(all_community_embeddings, memory, valid_nodes, index, index1,
           neighbors_unique, index_noself, index1_noself,
           neighbors_unique_noself, edge_weight, W1, b1, W2, b2, Ww, bw):
    del memory, index, index_noself, index1_noself, neighbors_unique_noself
    g1, g2, acew = _mm3(all_community_embeddings, W1, b1, W2, b2, Ww, bw)
    pidx = jnp.stack([neighbors_unique.astype(jnp.int32).reshape(NCHUNK, EC),
                      index1.astype(jnp.int32).reshape(NCHUNK, EC)], axis=1)
    idx13d = index1.astype(jnp.int32).reshape(DNCHUNK, DC // ESUB, ESUB)
    aggr = _edge_kernel(acew, pidx, edge_weight)
    deg = _deg_kernel(idx13d, edge_weight)
    vnp = jnp.pad(valid_nodes.astype(jnp.int32), (0, NPAD - N))
    return _combine_kernel(g1, g2, vnp, aggr, deg)


# speedup vs baseline: 1.2804x; 1.0946x over previous
import jax
import jax.numpy as jnp
from jax import lax
from jax.experimental import pallas as pl
from jax.experimental.pallas import tpu as pltpu
from jax.experimental.pallas import tpu_sc as plsc

N = 10000
E = 320000
D = 128
NC = 2
NS = 16
L = 16
NW = NC * NS
ESUB = 128

NPAD = 10240
RSTRIDE = NPAD // NS

EC = 128
NCHUNK = E // EC
KMAX = (NCHUNK + NW - 1) // NW
KMAX2 = (KMAX + 1) // 2

CC = 128
NCC = NPAD // CC
NFULL = N // CC
NREM = N - NFULL * CC
CKMAX = (NCC + NW - 1) // NW

_SC_MESH = dict(core_axis_name="c", subcore_axis_name="s",
                num_cores=NC, num_subcores=NS)


def _mm3_body(x_ref, w1_ref, b1_ref, w2_ref, b2_ref, ww_ref, bw_ref,
              g1_ref, g2_ref, gw_ref):
    x = x_ref[...]
    dn = (((1,), (1,)), ((), ()))
    g1_ref[...] = lax.dot_general(
        x, w1_ref[...], dn, preferred_element_type=jnp.float32) + b1_ref[...]
    g2_ref[...] = lax.dot_general(
        x, w2_ref[...], dn, preferred_element_type=jnp.float32) + b2_ref[...]
    gw_ref[...] = lax.dot_general(
        x, ww_ref[...], dn, preferred_element_type=jnp.float32) + bw_ref[...]


def _mm3(ace, W1, b1, W2, b2, Ww, bw):
    RB = 1000
    wspec = pl.BlockSpec((D, D), lambda i: (0, 0))
    bspec = pl.BlockSpec((1, D), lambda i: (0, 0))
    xspec = pl.BlockSpec((RB, D), lambda i: (i, 0))
    return pl.pallas_call(
        _mm3_body,
        grid=(N // RB,),
        in_specs=[xspec, wspec, bspec, wspec, bspec, wspec, bspec],
        out_specs=[xspec, xspec, xspec],
        out_shape=[jax.ShapeDtypeStruct((N, D), jnp.float32)] * 3,
    )(ace, W1, b1.reshape(1, D), W2, b2.reshape(1, D), Ww, bw.reshape(1, D))


def _edge_body(acew_hbm, pidx_hbm, ew_hbm, aggr_hbm, deg_hbm,
               acc, dacc, idxs0_v, idxs1_v, ew0_v, ew1_v, rows0_v, rows1_v,
               sem0, sem1):
    c = lax.axis_index("c")
    s = lax.axis_index("s")
    w = s * NC + c
    zero = jnp.zeros((L,), jnp.float32)

    def zrow(i, _):
        for j in range(D // L):
            rows0_v[i, pl.ds(j * L, L)] = zero
        return 0

    lax.fori_loop(0, EC, zrow, 0)
    r0 = s * RSTRIDE
    for off in range(0, RSTRIDE, EC):
        pltpu.sync_copy(rows0_v, acc.at[pl.ds(r0 + off, EC)])
    for off in range(0, RSTRIDE, D):
        pltpu.sync_copy(rows0_v.at[0], dacc.at[pl.ds(r0 + off, D)])
    plsc.subcore_barrier()

    def fetch(g, idxs_v, ew_v, rows_v, sem):
        pltpu.sync_copy(pidx_hbm.at[g], idxs_v)
        pltpu.sync_copy(ew_hbm.at[pl.ds(g * EC, EC)],
                        ew_v.at[pl.ds(0, EC)])
        pltpu.make_async_copy(acew_hbm.at[idxs_v.at[0]], rows_v, sem).start()

    def consume(idxs_v, ew_v, rows_v, sem):
        pltpu.make_async_copy(acew_hbm.at[idxs_v.at[0]], rows_v, sem).wait()

        def scale(i, _):
            wv = ew_v[pl.ds(i, L)]
            dv = jnp.full((L,), wv[0], jnp.float32)
            for j in range(D // L):
                sl = pl.ds(j * L, L)
                rows_v[i, sl] = rows_v[i, sl] * dv
            return 0

        lax.fori_loop(0, EC, scale, 0)
        pltpu.sync_copy(rows_v, acc.at[idxs_v.at[1]], add=True)
        pltpu.sync_copy(ew_v.at[pl.ds(0, EC)], dacc.at[idxs_v.at[1]],
                        add=True)

    fetch(w, idxs0_v, ew0_v, rows0_v, sem0)

    def chunk2(k2, _):
        ga = (2 * k2) * NW + w
        gb = ga + NW
        gc = gb + NW

        @pl.when(gb < NCHUNK)
        def _():
            fetch(gb, idxs1_v, ew1_v, rows1_v, sem1)

        @pl.when(ga < NCHUNK)
        def _():
            consume(idxs0_v, ew0_v, rows0_v, sem0)

        @pl.when(gc < NCHUNK)
        def _():
            fetch(gc, idxs0_v, ew0_v, rows0_v, sem0)

        @pl.when(gb < NCHUNK)
        def _():
            consume(idxs1_v, ew1_v, rows1_v, sem1)

        return 0

    lax.fori_loop(0, KMAX2, chunk2, 0)
    plsc.subcore_barrier()
    pltpu.sync_copy(acc.at[pl.ds(r0, RSTRIDE)],
                    aggr_hbm.at[c].at[pl.ds(r0, RSTRIDE)])
    pltpu.sync_copy(dacc.at[pl.ds(r0, RSTRIDE)],
                    deg_hbm.at[pl.ds(c * NPAD + r0, RSTRIDE)])


def _edge_kernel(acew, pidx, ew):
    mesh = plsc.VectorSubcoreMesh(**_SC_MESH)
    run = pl.kernel(
        _edge_body, mesh=mesh,
        out_type=[jax.ShapeDtypeStruct((NC, NPAD, D), jnp.float32),
                  jax.ShapeDtypeStruct((NC * NPAD,), jnp.float32)],
        scratch_types=[
            pltpu.MemorySpace.VMEM_SHARED((NPAD, D), jnp.float32),
            pltpu.MemorySpace.VMEM_SHARED((NPAD,), jnp.float32),
            pltpu.MemorySpace.VMEM((2, ESUB), jnp.int32),
            pltpu.MemorySpace.VMEM((2, ESUB), jnp.int32),
            pltpu.MemorySpace.VMEM((EC + L,), jnp.float32),
            pltpu.MemorySpace.VMEM((EC + L,), jnp.float32),
            pltpu.MemorySpace.VMEM((EC, D), jnp.float32),
            pltpu.MemorySpace.VMEM((EC, D), jnp.float32),
            pltpu.SemaphoreType.DMA,
            pltpu.SemaphoreType.DMA,
        ],
    )
    return run(acew, pidx, ew)


def _combine_body(g1_hbm, g2_hbm, vn_hbm, aggr_hbm, deg_hbm, out_hbm,
                  vn_v, h1_v, h2_v, a0_v, a1_v, d0_v, d1_v, out_v, sem):
    c = lax.axis_index("c")
    s = lax.axis_index("s")
    w = s * NC + c

    def chunk(k, _):
        cid = k * NW + w

        @pl.when(cid < NCC)
        def _():
            r0 = cid * CC
            pltpu.sync_copy(vn_hbm.at[pl.ds(r0, CC)], vn_v)
            dsc1 = pltpu.async_copy(g1_hbm.at[vn_v], h1_v, sem)
            dsc2 = pltpu.async_copy(g2_hbm.at[vn_v], h2_v, sem)
            dsc1.wait()
            dsc2.wait()
            pltpu.sync_copy(aggr_hbm.at[0].at[pl.ds(r0, CC)], a0_v)
            pltpu.sync_copy(aggr_hbm.at[1].at[pl.ds(r0, CC)], a1_v)
            pltpu.sync_copy(deg_hbm.at[pl.ds(r0, CC)],
                            d0_v.at[pl.ds(0, CC)])
            pltpu.sync_copy(deg_hbm.at[pl.ds(NPAD + r0, CC)],
                            d1_v.at[pl.ds(0, CC)])

            def row(i, _):
                d0 = d0_v[pl.ds(i, L)]
                d1 = d1_v[pl.ds(i, L)]
                dv = jnp.full((L,), d0[0] + d1[0], jnp.float32)
                for j in range(D // L):
                    sl = pl.ds(j * L, L)
                    out_v[i, sl] = (dv * h1_v[i, sl]
                                    + (a0_v[i, sl] + a1_v[i, sl]
                                       + h2_v[i, sl]))
                return 0

            lax.fori_loop(0, CC, row, 0)

            @pl.when(cid < NFULL)
            def _():
                pltpu.sync_copy(out_v, out_hbm.at[pl.ds(r0, CC)])

            @pl.when(cid == NFULL)
            def _():
                pltpu.sync_copy(out_v.at[pl.ds(0, NREM)],
                                out_hbm.at[pl.ds(r0, NREM)])
        return 0

    lax.fori_loop(0, CKMAX, chunk, 0)


def _combine_kernel(g1, g2, vn, aggr, deg):
    mesh = plsc.VectorSubcoreMesh(**_SC_MESH)
    run = pl.kernel(
        _combine_body, mesh=mesh,
        out_type=jax.ShapeDtypeStruct((N, D), jnp.float32),
        scratch_types=[
            pltpu.MemorySpace.VMEM((CC,), jnp.int32),
            pltpu.MemorySpace.VMEM((CC, D), jnp.float32),
            pltpu.MemorySpace.VMEM((CC, D), jnp.float32),
            pltpu.MemorySpace.VMEM((CC, D), jnp.float32),
            pltpu.MemorySpace.VMEM((CC, D), jnp.float32),
            pltpu.MemorySpace.VMEM((CC + L,), jnp.float32),
            pltpu.MemorySpace.VMEM((CC + L,), jnp.float32),
            pltpu.MemorySpace.VMEM((CC, D), jnp.float32),
            pltpu.SemaphoreType.DMA,
        ],
    )
    return run(g1, g2, vn, aggr, deg)


def kernel(all_community_embeddings, memory, valid_nodes, index, index1,
           neighbors_unique, index_noself, index1_noself,
           neighbors_unique_noself, edge_weight, W1, b1, W2, b2, Ww, bw):
    del memory, index, index_noself, index1_noself, neighbors_unique_noself
    g1, g2, acew = _mm3(all_community_embeddings, W1, b1, W2, b2, Ww, bw)
    pidx = jnp.stack([neighbors_unique.astype(jnp.int32).reshape(NCHUNK, EC),
                      index1.astype(jnp.int32).reshape(NCHUNK, EC)], axis=1)
    aggr, deg = _edge_kernel(acew, pidx, edge_weight)
    vnp = jnp.pad(valid_nodes.astype(jnp.int32), (0, NPAD - N))
    return _combine_kernel(g1, g2, vnp, aggr, deg)
